# baseline (speedup 1.0000x reference)
import functools

import jax
import jax.numpy as jnp
from jax import lax
from jax.experimental import pallas as pl
from jax.experimental.pallas import tpu as pltpu
from jax.experimental.pallas import tpu_sc as plsc

N_NODES = 10000
N_EDGES = 320000
D_FEAT_DIM = 128

NC, NS = 2, 16
NW = NC * NS
CHUNK = 40
ROWS_PER_W = 2 * N_EDGES // NW
NFULL = ROWS_PER_W // CHUNK
TAIL = ROWS_PER_W - NFULL * CHUNK
S = 5
G = 2

_mesh = plsc.VectorSubcoreMesh(core_axis_name="c", subcore_axis_name="s")


@functools.partial(
    pl.kernel,
    mesh=_mesh,
    out_type=jax.ShapeDtypeStruct((N_EDGES, 2 * D_FEAT_DIM), jnp.float32),
    scratch_types=[
        pltpu.VMEM((ROWS_PER_W,), jnp.int32),
        pltpu.VMEM((S, CHUNK, D_FEAT_DIM), jnp.float32),
        pltpu.VMEM_SHARED((N_NODES, D_FEAT_DIM), jnp.float32),
    ]
    + [pltpu.SemaphoreType.DMA] * (2 * S),
)
def _gather_rows(table, idx_all, out, idx_v, rows, table_sp, *sems):
    gsem = sems[:S]
    wsem = sems[S:]
    wid = lax.axis_index("s") * NC + lax.axis_index("c")
    sub = lax.axis_index("s")
    col = (wid // NS) * D_FEAT_DIM
    ebase = (wid % NS) * ROWS_PER_W

    STG = 624
    pltpu.sync_copy(
        table.at[pl.ds(sub * STG, STG)], table_sp.at[pl.ds(sub * STG, STG)]
    )

    @pl.when(sub == 0)
    def _():
        pltpu.sync_copy(
            table.at[pl.ds(NS * STG, N_NODES - NS * STG)],
            table_sp.at[pl.ds(NS * STG, N_NODES - NS * STG)],
        )

    pltpu.sync_copy(idx_all.at[pl.ds(wid * ROWS_PER_W, ROWS_PER_W)], idx_v)
    plsc.subcore_barrier()

    def gather_start(j, s):
        pltpu.make_async_copy(
            table_sp.at[idx_v.at[pl.ds(j * CHUNK, CHUNK)]], rows.at[s], gsem[s]
        ).start()

    def gather_wait(j, s):
        pltpu.make_async_copy(
            table_sp.at[idx_v.at[pl.ds(j * CHUNK, CHUNK)]], rows.at[s], gsem[s]
        ).wait()

    def write_start(j, s):
        pltpu.make_async_copy(
            rows.at[s],
            out.at[pl.ds(ebase + j * CHUNK, CHUNK), pl.ds(col, D_FEAT_DIM)],
            wsem[s],
        ).start()

    def write_wait(s):
        pltpu.make_async_copy(
            rows.at[s],
            out.at[pl.ds(0, CHUNK), pl.ds(0, D_FEAT_DIM)],
            wsem[s],
        ).wait()

    for k in range(G):
        gather_start(k, k)

    def body(i, _):
        for s in range(S):
            j = S * i + s
            sn = (s + G) % S
            gather_wait(j, s)
            write_start(j, s)
            @pl.when(j >= S - G)
            def _():
                write_wait(sn)

            @pl.when(j + G < NFULL)
            def _():
                gather_start(j + G, sn)

        return 0

    lax.fori_loop(0, NFULL // S, body, 0)

    for j in range(NFULL - (S - G), NFULL):
        write_wait(j % S)


def kernel(node_feature, edge_src, edge_dst):
    idx_all = jnp.concatenate(
        [edge_src.astype(jnp.int32), edge_dst.astype(jnp.int32)]
    )
    return _gather_rows(node_feature, idx_all)


# speedup vs baseline: 1.0387x; 1.0387x over previous
import functools

import jax
import jax.numpy as jnp
from jax import lax
from jax.experimental import pallas as pl
from jax.experimental.pallas import tpu as pltpu
from jax.experimental.pallas import tpu_sc as plsc

N_NODES = 10000
N_EDGES = 320000
D_FEAT_DIM = 128

NC, NS = 2, 16
NW = NC * NS
CHUNK = 40
ROWS_PER_W = 2 * N_EDGES // NW
NFULL = ROWS_PER_W // CHUNK
TAIL = ROWS_PER_W - NFULL * CHUNK
S = 5
G = 3

_mesh = plsc.VectorSubcoreMesh(core_axis_name="c", subcore_axis_name="s")


@functools.partial(
    pl.kernel,
    mesh=_mesh,
    out_type=jax.ShapeDtypeStruct((N_EDGES, 2 * D_FEAT_DIM), jnp.float32),
    scratch_types=[
        pltpu.VMEM((ROWS_PER_W,), jnp.int32),
        pltpu.VMEM((S, CHUNK, D_FEAT_DIM), jnp.float32),
        pltpu.VMEM_SHARED((N_NODES, D_FEAT_DIM), jnp.float32),
    ]
    + [pltpu.SemaphoreType.DMA] * (2 * S),
)
def _gather_rows(table, esrc, edst, out, idx_v, rows, table_sp, *sems):
    gsem = sems[:S]
    wsem = sems[S:]
    wid = lax.axis_index("s") * NC + lax.axis_index("c")
    sub = lax.axis_index("s")
    col = (wid // NS) * D_FEAT_DIM
    ebase = (wid % NS) * ROWS_PER_W

    STG = 624
    pltpu.sync_copy(
        table.at[pl.ds(sub * STG, STG)], table_sp.at[pl.ds(sub * STG, STG)]
    )

    @pl.when(sub == 0)
    def _():
        pltpu.sync_copy(
            table.at[pl.ds(NS * STG, N_NODES - NS * STG)],
            table_sp.at[pl.ds(NS * STG, N_NODES - NS * STG)],
        )

    @pl.when(wid < NS)
    def _():
        pltpu.sync_copy(esrc.at[pl.ds((wid % NS) * ROWS_PER_W, ROWS_PER_W)], idx_v)

    @pl.when(wid >= NS)
    def _():
        pltpu.sync_copy(edst.at[pl.ds((wid % NS) * ROWS_PER_W, ROWS_PER_W)], idx_v)

    plsc.subcore_barrier()

    def gather_start(j, s):
        pltpu.make_async_copy(
            table_sp.at[idx_v.at[pl.ds(j * CHUNK, CHUNK)]], rows.at[s], gsem[s]
        ).start()

    def gather_wait(j, s):
        pltpu.make_async_copy(
            table_sp.at[idx_v.at[pl.ds(j * CHUNK, CHUNK)]], rows.at[s], gsem[s]
        ).wait()

    def write_start(j, s):
        pltpu.make_async_copy(
            rows.at[s],
            out.at[pl.ds(ebase + j * CHUNK, CHUNK), pl.ds(col, D_FEAT_DIM)],
            wsem[s],
        ).start()

    def write_wait(s):
        pltpu.make_async_copy(
            rows.at[s],
            out.at[pl.ds(0, CHUNK), pl.ds(0, D_FEAT_DIM)],
            wsem[s],
        ).wait()

    for k in range(G):
        gather_start(k, k)

    def body(i, _):
        for s in range(S):
            j = S * i + s
            sn = (s + G) % S
            gather_wait(j, s)
            write_start(j, s)
            @pl.when(j >= S - G)
            def _():
                write_wait(sn)

            @pl.when(j + G < NFULL)
            def _():
                gather_start(j + G, sn)

        return 0

    lax.fori_loop(0, NFULL // S, body, 0)

    for j in range(NFULL - (S - G), NFULL):
        write_wait(j % S)


def kernel(node_feature, edge_src, edge_dst):
    return _gather_rows(
        node_feature, edge_src.astype(jnp.int32), edge_dst.astype(jnp.int32)
    )
